# baseline (speedup 1.0000x reference)
import functools

import jax
import jax.numpy as jnp
import numpy as np
from jax import lax
from jax.experimental import pallas as pl
from jax.experimental.pallas import tpu as pltpu

N = 10000
E = 320000
D = 128
NUM_BESSEL = 8
SH_DIM = 4
R_MAX = 5.0

EB = 1000
NB = 1000


def _embed_body(na_ref, we_ref, nf_ref):
    nf_ref[...] = jnp.dot(na_ref[...], we_ref[...],
                          preferred_element_type=jnp.float32)


def _embed(node_attrs, w_embed):
    return pl.pallas_call(
        _embed_body,
        grid=(N // NB,),
        in_specs=[
            pl.BlockSpec((NB, 8), lambda i: (i, 0)),
            pl.BlockSpec((8, D), lambda i: (0, 0)),
        ],
        out_specs=pl.BlockSpec((NB, D), lambda i: (i, 0)),
        out_shape=jax.ShapeDtypeStruct((N, D), jnp.float32),
    )(node_attrs, w_embed)


def _geom_body(ps_ref, pr_ref, sh_ref, wr_ref, ea_ref, r0_ref, r1_ref):
    v = pr_ref[...] - ps_ref[...] + sh_ref[...]
    x = v[:, 0:1]
    y = v[:, 1:2]
    z = v[:, 2:3]
    lengths = jnp.sqrt(x * x + y * y + z * z + 1e-8)
    s3 = np.sqrt(3.0).astype(np.float32)
    inv = s3 / lengths
    zero = jnp.zeros_like(lengths)
    one = jnp.ones_like(lengths)
    ea_ref[...] = jnp.concatenate(
        [one, y * inv, z * inv, x * inv, zero, zero, zero, zero], axis=1)
    n = lax.broadcasted_iota(jnp.int32, (1, NUM_BESSEL), 1).astype(
        jnp.float32) + 1.0
    pref = np.sqrt(2.0 / R_MAX).astype(np.float32)
    bess = pref * jnp.sin(n * (np.pi / R_MAX) * lengths) / lengths
    p = 6.0
    xs = lengths * (1.0 / R_MAX)
    x6 = xs * xs * xs
    x6 = x6 * x6
    f = (1.0 - ((p + 1.0) * (p + 2.0) / 2.0) * x6
         + p * (p + 2.0) * x6 * xs
         - (p * (p + 1.0) / 2.0) * x6 * xs * xs)
    cut = jnp.where(xs < 1.0, f, 0.0)
    ef = bess * cut
    wr = wr_ref[...]
    r0_ref[...] = jnp.dot(ef, wr[0], preferred_element_type=jnp.float32)
    r1_ref[...] = jnp.dot(ef, wr[1], preferred_element_type=jnp.float32)


def _geometry(ps8, pr8, sh8, w_radial):
    return pl.pallas_call(
        _geom_body,
        grid=(E // EB,),
        in_specs=[
            pl.BlockSpec((EB, 8), lambda i: (i, 0)),
            pl.BlockSpec((EB, 8), lambda i: (i, 0)),
            pl.BlockSpec((EB, 8), lambda i: (i, 0)),
            pl.BlockSpec((2, NUM_BESSEL, D), lambda i: (0, 0, 0)),
        ],
        out_specs=[
            pl.BlockSpec((EB, 8), lambda i: (i, 0)),
            pl.BlockSpec((EB, D), lambda i: (i, 0)),
            pl.BlockSpec((EB, D), lambda i: (i, 0)),
        ],
        out_shape=[
            jax.ShapeDtypeStruct((E, 8), jnp.float32),
            jax.ShapeDtypeStruct((E, D), jnp.float32),
            jax.ShapeDtypeStruct((E, D), jnp.float32),
        ],
    )(ps8, pr8, sh8, w_radial)


def _product_body(a_ref, nf_ref, wm_ref, wsc_ref, wp_ref, out_ref):
    a = a_ref[...]
    wm = wm_ref[...]
    am0 = jnp.dot(a[:, 0, :], wm, preferred_element_type=jnp.float32)
    am1 = jnp.dot(a[:, 1, :], wm, preferred_element_type=jnp.float32)
    am2 = jnp.dot(a[:, 2, :], wm, preferred_element_type=jnp.float32)
    am3 = jnp.dot(a[:, 3, :], wm, preferred_element_type=jnp.float32)
    b = am0 + am1 * am1 + am2 * am2 + am3 * am3
    out_ref[...] = (jnp.dot(b, wp_ref[...], preferred_element_type=jnp.float32)
                    + jnp.dot(nf_ref[...], wsc_ref[...],
                              preferred_element_type=jnp.float32))


def _product(a, node_feats, wm, wsc, wp):
    return pl.pallas_call(
        _product_body,
        grid=(N // NB,),
        in_specs=[
            pl.BlockSpec((NB, SH_DIM, D), lambda i: (i, 0, 0)),
            pl.BlockSpec((NB, D), lambda i: (i, 0)),
            pl.BlockSpec((D, D), lambda i: (0, 0)),
            pl.BlockSpec((D, D), lambda i: (0, 0)),
            pl.BlockSpec((D, D), lambda i: (0, 0)),
        ],
        out_specs=pl.BlockSpec((NB, D), lambda i: (i, 0)),
        out_shape=jax.ShapeDtypeStruct((N, D), jnp.float32),
    )(a, node_feats, wm, wsc, wp)


def kernel(node_attrs, atom_pos, shifts, W_embed, W_radial, W_msg, W_sc,
           W_prod, edge_index):
    sender = edge_index[0]
    receiver = edge_index[1]
    pos8 = jnp.pad(atom_pos, ((0, 0), (0, 5)))
    sh8 = jnp.pad(shifts, ((0, 0), (0, 5)))
    ps8 = pos8[sender]
    pr8 = pos8[receiver]
    ea8, r0, r1 = _geometry(ps8, pr8, sh8, W_radial)
    node_feats = _embed(node_attrs, W_embed)
    rs = (r0, r1)
    feats_list = []
    for i in range(2):
        m = node_feats[sender] * rs[i]
        a = jax.vmap(
            lambda l: jax.ops.segment_sum(m * ea8[:, l][:, None], receiver,
                                          num_segments=N),
            out_axes=1)(jnp.arange(SH_DIM))
        node_feats = _product(a, node_feats, W_msg[i], W_sc[i], W_prod[i])
        feats_list.append(node_feats)
    return jnp.concatenate(feats_list, axis=-1)
